# baseline (speedup 1.0000x reference)
import functools

import jax
import jax.numpy as jnp
from jax import lax
from jax.experimental import pallas as pl
from jax.experimental.pallas import tpu as pltpu
from jax.experimental.pallas import tpu_sc as plsc

R = 256
C = 100000
N = 16384
NC, NS, L = 2, 16, 16
NW = NC * NS
CW = 3136
RB = 224
NBLK = CW // RB
CHUNK = 128
NCH = 25
LSZ = NCH * CHUNK


def _make_sc_kernel(interpret=False):
    mesh = plsc.VectorSubcoreMesh(core_axis_name="c", subcore_axis_name="s",
                                  num_cores=NC, num_subcores=NS)

    @functools.partial(
        pl.kernel,
        out_type=jax.ShapeDtypeStruct((C, R), jnp.float32),
        mesh=mesh,
        scratch_types=[
            pltpu.VMEM((N,), jnp.int32),
            pltpu.VMEM((CW,), jnp.int32),
            pltpu.VMEM((LSZ,), jnp.int32),
            pltpu.VMEM((LSZ,), jnp.int32),
            pltpu.VMEM((NCH, CHUNK), jnp.int32),
            pltpu.VMEM((CHUNK, R), jnp.float32),
            pltpu.VMEM((RB, R), jnp.float32),
            pltpu.VMEM((L,), jnp.int32),
            pltpu.SemaphoreType.DMA,
            pltpu.SemaphoreType.DMA,
        ],
        compiler_params=pltpu.CompilerParams(needs_layout_passes=False),
        interpret=interpret,
    )
    def k(in_hbm, idx_hbm, val_hbm, out_hbm,
          idx_v, W, jlist, rlist, sidx, P, buf, nbuf, gsem, ssem):
        wid = lax.axis_index("c") * NS + lax.axis_index("s")
        row0 = jnp.minimum(wid * CW, C - CW)
        lane = lax.iota(jnp.int32, L)
        four = jnp.full((L,), 4, jnp.int32)

        pltpu.sync_copy(idx_hbm, idx_v)

        def init_body(t, c):
            W[pl.ds(t * L, L)] = jnp.full((L,), -1, jnp.int32)
            return c
        lax.fori_loop(0, CW // L, init_body, 0)

        def scan_body(t, c):
            iv = idx_v[pl.ds(t * L, L)]
            jv = t * L + lane
            key = iv * L + lane
            skey, sj = plsc.sort_key_val(key, jv)
            srv = lax.shift_right_logical(skey, four)
            nbuf[...] = srv
            nxt = plsc.load_gather(nbuf, [jnp.minimum(lane + 1, L - 1)])
            rl = srv - row0
            valid = (rl >= 0) & (rl < CW)
            keep = ((srv != nxt) | (lane == L - 1)) & valid
            r_safe = jnp.clip(rl, 0, CW - 1)
            plsc.store_scatter(W, [r_safe], sj, mask=keep)
            return c
        lax.fori_loop(0, N // L, scan_body, 0)

        def comp_body(t, off):
            wv = W[pl.ds(t * L, L)]
            m = wv >= 0
            rv = t * L + lane
            plsc.store_compressed(rlist.at[pl.ds(off, L)], rv, mask=m)
            plsc.store_compressed(jlist.at[pl.ds(off, L)], wv, mask=m)
            return off + jnp.sum(jnp.where(m, 1, 0))
        nk = lax.fori_loop(0, CW // L, comp_body, jnp.int32(0))

        nch = (nk + (CHUNK - 1)) // CHUNK
        lastp = jnp.maximum(nk - 1, 0)
        lastr = plsc.load_gather(rlist, [jnp.zeros((L,), jnp.int32) + lastp])
        lastj = plsc.load_gather(jlist, [jnp.zeros((L,), jnp.int32) + lastp])

        def pad_tail(t, c):
            pos = t * L + lane
            m = (pos >= nk) & (pos < nch * CHUNK)
            plsc.store_scatter(rlist, [jnp.clip(pos, 0, LSZ - 1)], lastr,
                               mask=m)
            plsc.store_scatter(jlist, [jnp.clip(pos, 0, LSZ - 1)], lastj,
                               mask=m)
            return c
        lax.fori_loop(lastp // L, jnp.minimum(lastp // L + (CHUNK // L) + 1,
                                              LSZ // L), pad_tail, 0)

        def repack_body(t, c):
            q = t // (CHUNK // L)
            p = t % (CHUNK // L)
            rv = rlist[pl.ds(t * L, L)] + row0
            plsc.store_scatter(sidx, [jnp.zeros((L,), jnp.int32) + q,
                                      p * L + lane], rv)
            return c
        lax.fori_loop(0, nch * (CHUNK // L), repack_body, 0)

        pltpu.sync_copy(in_hbm.at[pl.ds(row0, CW), :],
                        out_hbm.at[pl.ds(row0, CW), :])

        plsc.subcore_barrier()

        def patch_body(q, c):
            pltpu.async_copy(val_hbm.at[jlist.at[pl.ds(q * CHUNK, CHUNK)]],
                             P, gsem).wait()
            pltpu.async_copy(P, out_hbm.at[sidx.at[q]], ssem).wait()
            return c
        lax.fori_loop(0, nch, patch_body, 0)

    return k


def kernel(input, index, value):
    index = index.astype(jnp.int32)
    outt = _make_sc_kernel()(input.T, index, value.T)
    return outt.T


# speedup vs baseline: 22.1405x; 22.1405x over previous
import functools

import jax
import jax.numpy as jnp
from jax import lax
from jax.experimental import pallas as pl
from jax.experimental.pallas import tpu as pltpu
from jax.experimental.pallas import tpu_sc as plsc

R = 256
C = 100000
N = 16384
NC, NS, L = 2, 16, 16
NW = NC * NS
CW = 3136
RB = 112
NBLK = CW // RB
CHUNK = 128
NCH = 25
LSZ = NCH * CHUNK


def _make_sc_kernel(interpret=False):
    mesh = plsc.VectorSubcoreMesh(core_axis_name="c", subcore_axis_name="s",
                                  num_cores=NC, num_subcores=NS)

    @functools.partial(
        pl.kernel,
        out_type=jax.ShapeDtypeStruct((C, R), jnp.float32),
        mesh=mesh,
        scratch_types=[
            pltpu.VMEM((N,), jnp.int32),
            pltpu.VMEM((CW,), jnp.int32),
            pltpu.VMEM((LSZ,), jnp.int32),
            pltpu.VMEM((LSZ,), jnp.int32),
            pltpu.VMEM((NCH, CHUNK), jnp.int32),
            pltpu.VMEM((CHUNK, R), jnp.float32),
            pltpu.VMEM((RB, R), jnp.float32),
            pltpu.VMEM((RB, R), jnp.float32),
            pltpu.VMEM((L,), jnp.int32),
            pltpu.SemaphoreType.DMA,
            pltpu.SemaphoreType.DMA,
            pltpu.SemaphoreType.DMA,
            pltpu.SemaphoreType.DMA,
            pltpu.SemaphoreType.DMA,
            pltpu.SemaphoreType.DMA,
        ],
        compiler_params=pltpu.CompilerParams(needs_layout_passes=False),
        interpret=interpret,
    )
    def k(in_hbm, idx_hbm, val_hbm, out_hbm,
          idx_v, W, jlist, rlist, sidx, P, buf0, buf1, nbuf,
          gsem, ssem, si0, si1, so0, so1):
        wid = lax.axis_index("c") * NS + lax.axis_index("s")
        row0 = jnp.minimum(wid * CW, C - CW)
        lane = lax.iota(jnp.int32, L)
        four = jnp.full((L,), 4, jnp.int32)

        pltpu.sync_copy(idx_hbm, idx_v)

        def init_body(t, c):
            W[pl.ds(t * L, L)] = jnp.full((L,), -1, jnp.int32)
            return c
        lax.fori_loop(0, CW // L, init_body, 0)

        def scan_body(t, c):
            iv = idx_v[pl.ds(t * L, L)]
            jv = t * L + lane
            key = iv * L + lane
            skey, sj = plsc.sort_key_val(key, jv)
            srv = lax.shift_right_logical(skey, four)
            nbuf[...] = srv
            nxt = plsc.load_gather(nbuf, [jnp.minimum(lane + 1, L - 1)])
            rl = srv - row0
            valid = (rl >= 0) & (rl < CW)
            keep = ((srv != nxt) | (lane == L - 1)) & valid
            r_safe = jnp.clip(rl, 0, CW - 1)
            plsc.store_scatter(W, [r_safe], sj, mask=keep)
            return c

        bufs = (buf0, buf1)
        sis = (si0, si1)
        sos = (so0, so1)
        nv = N // L
        spb = -(-nv // NBLK)

        def blk(b):
            return in_hbm.at[pl.ds(row0 + b * RB, RB), :]

        def oblk(b):
            return out_hbm.at[pl.ds(row0 + b * RB, RB), :]

        din = [None, None]
        dout = [None, None]
        din[0] = pltpu.async_copy(blk(0), bufs[0], sis[0])
        for b in range(NBLK):
            B = b % 2
            lo, hi = min(b * spb, nv), min((b + 1) * spb, nv)
            if lo < hi:
                lax.fori_loop(lo, hi, scan_body, 0)
            din[B].wait()
            dout[B] = pltpu.async_copy(bufs[B], oblk(b), sos[B])
            if b + 1 < NBLK:
                if b >= 1:
                    dout[1 - B].wait()
                din[1 - B] = pltpu.async_copy(blk(b + 1), bufs[1 - B],
                                              sis[1 - B])
        dout[(NBLK - 2) % 2].wait()
        dout[(NBLK - 1) % 2].wait()

        def comp_body(t, off):
            wv = W[pl.ds(t * L, L)]
            m = wv >= 0
            rv = t * L + lane
            plsc.store_compressed(rlist.at[pl.ds(off, L)], rv, mask=m)
            plsc.store_compressed(jlist.at[pl.ds(off, L)], wv, mask=m)
            return off + jnp.sum(jnp.where(m, 1, 0))
        nk = lax.fori_loop(0, CW // L, comp_body, jnp.int32(0))

        nch = (nk + (CHUNK - 1)) // CHUNK
        lastp = jnp.maximum(nk - 1, 0)
        lastr = plsc.load_gather(rlist, [jnp.zeros((L,), jnp.int32) + lastp])
        lastj = plsc.load_gather(jlist, [jnp.zeros((L,), jnp.int32) + lastp])

        def pad_tail(t, c):
            pos = t * L + lane
            m = (pos >= nk) & (pos < nch * CHUNK)
            plsc.store_scatter(rlist, [jnp.clip(pos, 0, LSZ - 1)], lastr,
                               mask=m)
            plsc.store_scatter(jlist, [jnp.clip(pos, 0, LSZ - 1)], lastj,
                               mask=m)
            return c
        lax.fori_loop(lastp // L, jnp.minimum(lastp // L + (CHUNK // L) + 1,
                                              LSZ // L), pad_tail, 0)

        def repack_body(t, c):
            q = t // (CHUNK // L)
            p = t % (CHUNK // L)
            rv = rlist[pl.ds(t * L, L)] + row0
            plsc.store_scatter(sidx, [jnp.zeros((L,), jnp.int32) + q,
                                      p * L + lane], rv)
            return c
        lax.fori_loop(0, nch * (CHUNK // L), repack_body, 0)

        plsc.subcore_barrier()

        def patch_body(q, c):
            pltpu.async_copy(val_hbm.at[jlist.at[pl.ds(q * CHUNK, CHUNK)]],
                             P, gsem).wait()
            pltpu.async_copy(P, out_hbm.at[sidx.at[q]], ssem).wait()
            return c
        lax.fori_loop(0, nch, patch_body, 0)

    return k


def kernel(input, index, value):
    index = index.astype(jnp.int32)
    outt = _make_sc_kernel()(input.T, index, value.T)
    return outt.T
